# Initial kernel scaffold; baseline (speedup 1.0000x reference)
import jax
import jax.numpy as jnp
from jax.experimental import pallas as pl


def kernel(x, edge_index, batch_idx, rnafm, hand_feat, params):
    raise NotImplementedError("write your pallas kernel here")


# speedup vs baseline: 1.0000x; 1.0000x over previous
import jax
import jax.numpy as jnp
from jax.experimental import pallas as pl

H = 4
D = 64
B = 64


def _gelu(v):
    return jax.nn.gelu(v, approximate=False)


def _gat(x, src, dst, W, a_s, a_d, b):
    n = x.shape[0]
    h = (x @ W).reshape(n, H, D)
    a_src = (h * a_s[None]).sum(-1)
    a_dst = (h * a_d[None]).sum(-1)
    e = a_src[src] + a_dst[dst]
    e = jnp.where(e > 0, e, 0.2 * e)
    emax = jax.ops.segment_max(e, dst, num_segments=n)
    emax = jnp.where(jnp.isfinite(emax), emax, 0.0)
    ee = jnp.exp(e - emax[dst])
    denom = jax.ops.segment_sum(ee, dst, num_segments=n)
    alpha = ee / (denom[dst] + 1e-16)
    out = jax.ops.segment_sum(h[src] * alpha[:, :, None], dst, num_segments=n)
    return out.mean(axis=1) + b


def _id_kernel(x_ref, o_ref):
    o_ref[...] = x_ref[...]


def kernel(x, edge_index, batch_idx, rnafm, hand_feat, params):
    p = params
    n = x.shape[0]
    loop = jnp.arange(n, dtype=edge_index.dtype)
    src = jnp.concatenate([edge_index[0], loop])
    dst = jnp.concatenate([edge_index[1], loop])
    h = _gelu(_gat(x, src, dst, p['W1'], p['as1'], p['ad1'], p['b1']))
    h = _gelu(_gat(h, src, dst, p['W2'], p['as2'], p['ad2'], p['b2']))
    h = _gelu(_gat(h, src, dst, p['W3'], p['as3'], p['ad3'], p['b3']))
    attn = jax.nn.softmax(h @ p['pw'] + p['pb'], axis=0)
    weighted = h * attn
    pooled = jax.ops.segment_sum(weighted, batch_idx, num_segments=B)
    fused = jnp.concatenate([pooled, rnafm, hand_feat], axis=-1)
    z = _gelu(fused @ p['e1w'] + p['e1b'])
    mu = z.mean(-1, keepdims=True)
    var = ((z - mu) ** 2).mean(-1, keepdims=True)
    z = (z - mu) / jnp.sqrt(var + 1e-5) * p['lng'] + p['lnb']
    shared = _gelu(z @ p['e2w'] + p['e2b'])
    shared = pl.pallas_call(
        _id_kernel,
        out_shape=jax.ShapeDtypeStruct(shared.shape, shared.dtype),
    )(shared)
    binary_logit = (shared @ p['bw'] + p['bb'])[:, 0]
    per_enzyme = tuple((_gelu(shared @ a['w1'] + a['b1']) @ a['w2'] + a['b2'])[:, 0] for a in p['adapters'])
    cls = _gelu(shared @ p['cw1'] + p['cb1']) @ p['cw2'] + p['cb2']
    return (binary_logit, per_enzyme, cls)


# speedup vs baseline: 54.6332x; 54.6331x over previous
import functools

import jax
import jax.numpy as jnp
from jax import lax
from jax.experimental import pallas as pl
from jax.experimental.pallas import tpu as pltpu
from jax.experimental.pallas import tpu_sc as plsc

N = 50000
E = 800000
EP = E + N
B = 64
H = 4
D = 64
HD = H * D

NC = 2
NS = 16
NW = NC * NS
R = 64
BR = 512
NBLK = 784
N_BUF = NBLK * R
GRID = N_BUF // BR
K = 64
EPK = ((EP + K + 7) // 8) * 8
ADST_MAXSTART = N_BUF - ((NBLK + NW - 1) // NW) * R


def _gelu(v):
    return 0.5 * v * (1.0 + lax.erf(v * 0.7071067811865476))


def _tc_layer_body(first, xin_ref, w_ref, as_ref, ad_ref, bprev_ref,
                   h_ref, adst_ref, gmax_ref, mx_ref):
    i = pl.program_id(0)
    if first:
        x = xin_ref[...]
    else:
        accb = xin_ref[...]
        acc = accb[:, :HD].reshape(BR, H, D)
        den = accb[:, HD:HD + H].reshape(BR, H, 1)
        x = _gelu((acc / (den + 1e-16)).mean(axis=1) + bprev_ref[...])
    h = jnp.dot(x, w_ref[...], preferred_element_type=jnp.float32)
    hr = h.reshape(BR, H, D)
    a_src = (hr * as_ref[...][None]).sum(-1)
    a_dst = (hr * ad_ref[...][None]).sum(-1)
    pad = jnp.zeros((BR, 16 - H), jnp.float32)
    h_ref[...] = jnp.concatenate(
        [h, a_src, pad, jnp.zeros((BR, 112), jnp.float32)], axis=-1)
    adst_ref[...] = jnp.concatenate([a_dst, pad], axis=-1)

    rows = i * BR + lax.broadcasted_iota(jnp.int32, (BR, 1), 0)
    valid = rows < N
    neg = jnp.float32(-1e30)
    asm = jnp.max(jnp.where(valid, a_src, neg), axis=0)
    adm = jnp.max(jnp.where(valid, a_dst, neg), axis=0)
    lane = lax.broadcasted_iota(jnp.int32, (1, 128), 1)
    asv = jnp.where(lane < H, jnp.concatenate(
        [asm, jnp.zeros((128 - H,), jnp.float32)])[None, :], neg)
    adv = jnp.where(lane < H, jnp.concatenate(
        [adm, jnp.zeros((128 - H,), jnp.float32)])[None, :], neg)

    @pl.when(i == 0)
    def _():
        mx_ref[...] = jnp.full((2, 128), neg, jnp.float32)

    mx_ref[0:1, :] = jnp.maximum(mx_ref[0:1, :], asv)
    mx_ref[1:2, :] = jnp.maximum(mx_ref[1:2, :], adv)

    @pl.when(i == pl.num_programs(0) - 1)
    def _():
        g = jnp.maximum(mx_ref[0:1, :] + mx_ref[1:2, :], 0.0)
        gmax_ref[...] = jnp.where(lane < H, g, 0.0).reshape(128)


def _tc_layer(xin, w, a_s, a_d, bprev, first):
    in_w = xin.shape[1]
    body = functools.partial(_tc_layer_body, first)
    return pl.pallas_call(
        body,
        grid=(GRID,),
        in_specs=[
            pl.BlockSpec((BR, in_w), lambda i: (i, 0)),
            pl.BlockSpec(w.shape, lambda i: (0, 0)),
            pl.BlockSpec((H, D), lambda i: (0, 0)),
            pl.BlockSpec((H, D), lambda i: (0, 0)),
            pl.BlockSpec((1, D), lambda i: (0, 0)),
        ],
        out_specs=[
            pl.BlockSpec((BR, HD + 128), lambda i: (i, 0)),
            pl.BlockSpec((BR, 16), lambda i: (i, 0)),
            pl.BlockSpec((128,), lambda i: (0,)),
        ],
        out_shape=[
            jax.ShapeDtypeStruct((N_BUF, HD + 128), jnp.float32),
            jax.ShapeDtypeStruct((N_BUF, 16), jnp.float32),
            jax.ShapeDtypeStruct((128,), jnp.float32),
        ],
        scratch_shapes=[pltpu.VMEM((2, 128), jnp.float32)],
    )(xin, w, a_s, a_d, bprev)


def _sc_edge_body(h_hbm, adst_hbm, gmax_hbm, src_hbm, dst_hbm,
                  meta_hbm, acc_hbm, den_hbm,
                  meta_v, gmax_v, idx_v0, idx_v1, dstb_v0, dstb_v1,
                  rows_v0, rows_v1, adst_v, stacc_v, stden_v,
                  gsem0, gsem1, isem0, isem1):
    cid = lax.axis_index("c")
    sid = lax.axis_index("s")
    wid = sid * NC + cid

    pltpu.sync_copy(meta_hbm.at[wid], meta_v)
    pltpu.sync_copy(gmax_hbm.at[pl.ds(0, 16)], gmax_v)
    mv = meta_v[...]
    e_start = mv[0]
    nb = mv[1]
    n_lo = mv[2]
    n_hi = mv[3]
    e_lo = mv[4]
    e_hi = mv[5]
    pltpu.sync_copy(adst_hbm.at[pl.ds(pl.multiple_of(n_lo, 8), R)],
                    adst_v.at[pl.ds(0, R)])
    pltpu.sync_copy(
        adst_hbm.at[pl.ds(pl.multiple_of(
            jnp.minimum(n_lo + R, N_BUF - R), 8), R)],
        adst_v.at[pl.ds(R, R)])
    gmax = gmax_v[...]
    lane = lax.broadcasted_iota(jnp.int32, (16,), 0)
    _gd = lax.GatherDimensionNumbers(
        offset_dims=(), collapsed_slice_dims=(0,), start_index_map=(0,))
    sidx = [(lane * 0 + hh).reshape(16, 1) for hh in range(4)]

    def splat(vec, hh):
        return lax.gather(vec, sidx[hh], _gd, (1,),
                          mode=lax.GatherScatterMode.PROMISE_IN_BOUNDS)

    zero = jnp.zeros((16,), jnp.float32)

    def do_flush(b0, refill):
        b0a = pl.multiple_of(b0, 8)
        half = pl.multiple_of((b0 // R % 2) * R, 8)
        pltpu.sync_copy(stacc_v.at[pl.ds(half, R)], acc_hbm.at[pl.ds(b0a, R)])
        pltpu.sync_copy(stden_v.at[pl.ds(half, R)], den_hbm.at[pl.ds(b0a, R)])
        if refill:
            nxt = pl.multiple_of(jnp.minimum(b0 + 2 * R, N_BUF - R), 8)
            pltpu.sync_copy(adst_hbm.at[pl.ds(nxt, R)],
                            adst_v.at[pl.ds(half, R)])
        return b0 + R

    def eoff(b):
        return pl.multiple_of(jnp.minimum(e_start + b * K, EPK - K), 8)

    def load_idx(b, idxr, dstr, sem):
        o = eoff(b)
        pltpu.async_copy(src_hbm.at[pl.ds(o, K)], idxr, sem)
        pltpu.async_copy(dst_hbm.at[pl.ds(o, K)], dstr, sem)

    def wait_idx(idxr, dstr, sem):
        pltpu.make_async_copy(src_hbm.at[pl.ds(0, K)], idxr, sem).wait()
        pltpu.make_async_copy(dst_hbm.at[pl.ds(0, K)], dstr, sem).wait()

    def process(b, dstb_v, rows_v, carry):
        prev, base, den, accs = carry
        eb = e_start + b * K

        first16 = dstb_v[pl.ds(0, 16)]
        fvalid = jnp.logical_and(eb >= e_lo, eb < e_hi)
        dstf = jnp.where(fvalid, jnp.clip(first16[0], n_lo, n_hi - 1), prev)
        base = lax.cond(dstf - base >= R,
                        lambda b0: do_flush(b0, True), lambda b0: b0, base)
        carry = (prev, base, den, accs)

        def group_body(g, c):
            dst16 = dstb_v[pl.ds(g * 16, 16)]
            for j2 in range(16):
                prev, base, den, accs = c
                j = g * 16 + j2
                eg = eb + j
                valid = jnp.logical_and(eg >= e_lo, eg < e_hi)
                dstc = jnp.where(valid, jnp.clip(dst16[j2], n_lo, n_hi - 1),
                                 prev)
                changed = dstc != prev
                rowoff = dstc % (2 * R)

                row = [rows_v[j, pl.ds(16 * v, 16)] for v in range(16)]
                a_sv = rows_v[j, pl.ds(HD, 16)]
                a_d = adst_v[rowoff]
                ev = a_sv + a_d
                ev = jnp.where(ev > 0, ev, 0.2 * ev)
                ee = jnp.exp(ev - gmax)
                ee = jnp.where(valid, ee, zero)

                den = jnp.where(changed, zero, den) + ee
                stden_v[rowoff] = den
                sp = [splat(ee, hh) for hh in range(4)]
                new_accs = []
                for v in range(16):
                    a = jnp.where(changed, zero, accs[v]) + sp[v // 4] * row[v]
                    stacc_v[rowoff, pl.ds(16 * v, 16)] = a
                    new_accs.append(a)
                c = (dstc, base, den, tuple(new_accs))
            return c

        return lax.fori_loop(0, K // 16, group_body, carry)

    load_idx(0, idx_v0, dstb_v0, isem0)
    wait_idx(idx_v0, dstb_v0, isem0)
    pltpu.async_copy(h_hbm.at[idx_v0], rows_v0, gsem0)
    load_idx(1, idx_v1, dstb_v1, isem1)

    def pair_body(i, carry):
        b0 = 2 * i
        wait_idx(idx_v1, dstb_v1, isem1)
        pltpu.async_copy(h_hbm.at[idx_v1], rows_v1, gsem1)
        pltpu.make_async_copy(h_hbm.at[idx_v0], rows_v0, gsem0).wait()
        carry = process(b0, dstb_v0, rows_v0, carry)
        load_idx(b0 + 2, idx_v0, dstb_v0, isem0)
        wait_idx(idx_v0, dstb_v0, isem0)
        pltpu.async_copy(h_hbm.at[idx_v0], rows_v0, gsem0)
        pltpu.make_async_copy(h_hbm.at[idx_v1], rows_v1, gsem1).wait()
        carry = process(b0 + 1, dstb_v1, rows_v1, carry)
        load_idx(b0 + 3, idx_v1, dstb_v1, isem1)
        return carry

    init = (n_lo, n_lo, zero, tuple(zero for _ in range(16)))
    npairs = (nb + 1) // 2
    prev, base, den, accs = lax.fori_loop(0, npairs, pair_body, init)

    pltpu.make_async_copy(h_hbm.at[idx_v0], rows_v0, gsem0).wait()
    wait_idx(idx_v1, dstb_v1, isem1)

    base = lax.cond(prev - base >= R,
                    lambda b0: do_flush(b0, False), lambda b0: b0, base)
    do_flush(base, False)


def _sc_edge(h_t, adst, gmax, srcs, dsts, meta):
    mesh = plsc.VectorSubcoreMesh(core_axis_name="c", subcore_axis_name="s")
    f = pl.kernel(
        _sc_edge_body,
        out_type=[
            jax.ShapeDtypeStruct((N_BUF, HD), jnp.float32),
            jax.ShapeDtypeStruct((N_BUF, 16), jnp.float32),
        ],
        mesh=mesh,
        scratch_types=[
            pltpu.VMEM((16,), jnp.int32),
            pltpu.VMEM((16,), jnp.float32),
            pltpu.VMEM((K,), jnp.int32),
            pltpu.VMEM((K,), jnp.int32),
            pltpu.VMEM((K,), jnp.int32),
            pltpu.VMEM((K,), jnp.int32),
            pltpu.VMEM((K, HD + 128), jnp.float32),
            pltpu.VMEM((K, HD + 128), jnp.float32),
            pltpu.VMEM((2 * R, 16), jnp.float32),
            pltpu.VMEM((2 * R, HD), jnp.float32),
            pltpu.VMEM((2 * R, 16), jnp.float32),
            pltpu.SemaphoreType.DMA,
            pltpu.SemaphoreType.DMA,
            pltpu.SemaphoreType.DMA,
            pltpu.SemaphoreType.DMA,
        ],
    )
    return f(h_t, adst, gmax, srcs, dsts, meta)


def _tc_tail_body(acc_ref, bidx_ref, b3_ref, pw_ref, rnafm_ref, hand_ref,
                  e1w_ref, e1b_ref, lng_ref, lnb_ref, e2w_ref, e2b_ref,
                  bw_ref, aw1_ref, ab1_ref, aw2_ref, cw1_ref, cb1_ref,
                  cw2_ref, bias_ref,
                  bin_ref, enz_ref, cls_ref, mz_ref, accp_ref):
    i = pl.program_id(0)
    accb = acc_ref[...]
    acc = accb[:, :HD].reshape(BR, H, D)
    den = accb[:, HD:HD + H].reshape(BR, H, 1)
    x = _gelu((acc / (den + 1e-16)).mean(axis=1) + b3_ref[...])

    rows = i * BR + lax.broadcasted_iota(jnp.int32, (BR, 1), 0)
    valid = rows < N
    x = jnp.where(valid, x, 0.0)
    s = jnp.sum(x * pw_ref[...], axis=-1, keepdims=True)
    s = jnp.where(valid, s, jnp.float32(-1e30))

    @pl.when(i == 0)
    def _():
        mz_ref[0] = jnp.float32(-1e30)
        mz_ref[1] = jnp.float32(0.0)
        accp_ref[...] = jnp.zeros((B, D), jnp.float32)

    m_old = mz_ref[0]
    bm = jnp.max(s)
    m_new = jnp.maximum(m_old, bm)
    c = jnp.exp(m_old - m_new)
    w = jnp.exp(s - m_new)
    w = jnp.where(valid, w, 0.0)
    mz_ref[0] = m_new
    mz_ref[1] = mz_ref[1] * c + jnp.sum(w)
    onehot_t = (bidx_ref[0] == lax.broadcasted_iota(
        jnp.int32, (B, BR), 0)).astype(jnp.float32)
    contrib = jnp.dot(onehot_t, w * x, preferred_element_type=jnp.float32)
    accp_ref[...] = accp_ref[...] * c + contrib

    @pl.when(i == pl.num_programs(0) - 1)
    def _():
        pooled = accp_ref[...] / mz_ref[1]
        fused = jnp.concatenate(
            [pooled, rnafm_ref[...], hand_ref[...]], axis=-1)
        z = _gelu(jnp.dot(fused, e1w_ref[...],
                          preferred_element_type=jnp.float32) + e1b_ref[...])
        mu = z.mean(-1, keepdims=True)
        var = ((z - mu) ** 2).mean(-1, keepdims=True)
        z = (z - mu) / jnp.sqrt(var + 1e-5) * lng_ref[...] + lnb_ref[...]
        shared = _gelu(jnp.dot(z, e2w_ref[...],
                               preferred_element_type=jnp.float32) + e2b_ref[...])
        bias = bias_ref[...]
        bin_ref[...] = jnp.dot(shared, bw_ref[...],
                               preferred_element_type=jnp.float32) + bias[:, 0:8]
        za = _gelu(jnp.dot(shared, aw1_ref[...],
                           preferred_element_type=jnp.float32) + ab1_ref[...])
        enz_ref[...] = jnp.dot(za, aw2_ref[...],
                               preferred_element_type=jnp.float32) + bias[:, 8:16]
        zc = _gelu(jnp.dot(shared, cw1_ref[...],
                           preferred_element_type=jnp.float32) + cb1_ref[...])
        cls_ref[...] = jnp.dot(zc, cw2_ref[...],
                               preferred_element_type=jnp.float32) + bias[:, 16:24]


def _tc_tail(acc3, bidx3d, b3, pwr, rnafm, hand, e1w, e1b, lng, lnb,
             e2w, e2b, bw8, aw1, ab1, aw2, cw1, cb1, cw2, bias):
    whole = lambda shape: pl.BlockSpec(shape, lambda i: tuple(0 for _ in shape))
    return pl.pallas_call(
        _tc_tail_body,
        grid=(GRID,),
        in_specs=[
            pl.BlockSpec((BR, HD + 16), lambda i: (i, 0)),
            pl.BlockSpec((1, 1, BR), lambda i: (i, 0, 0)),
            whole((1, D)), whole((1, D)),
            whole((B, 640)), whole((B, 40)),
            whole((744, 256)), whole((1, 256)), whole((1, 256)), whole((1, 256)),
            whole((256, 128)), whole((1, 128)),
            whole((128, 8)), whole((128, 160)), whole((1, 160)), whole((160, 8)),
            whole((128, 64)), whole((1, 64)), whole((64, 8)),
            whole((1, 128)),
        ],
        out_specs=[
            whole((B, 8)), whole((B, 8)), whole((B, 8)),
        ],
        out_shape=[
            jax.ShapeDtypeStruct((B, 8), jnp.float32),
            jax.ShapeDtypeStruct((B, 8), jnp.float32),
            jax.ShapeDtypeStruct((B, 8), jnp.float32),
        ],
        scratch_shapes=[
            pltpu.SMEM((2,), jnp.float32),
            pltpu.VMEM((B, D), jnp.float32),
        ],
    )(acc3, bidx3d, b3, pwr, rnafm, hand, e1w, e1b, lng, lnb, e2w, e2b,
      bw8, aw1, ab1, aw2, cw1, cb1, cw2, bias)


def kernel(x, edge_index, batch_idx, rnafm, hand_feat, params):
    p = params
    f32 = jnp.float32

    loop = jnp.arange(N, dtype=edge_index.dtype)
    src = jnp.concatenate([edge_index[0], loop])
    dst = jnp.concatenate([edge_index[1], loop])
    dst_s, src_s = lax.sort_key_val(dst, src)
    src_p = jnp.concatenate([src_s, jnp.zeros((EPK - EP,), jnp.int32)])
    dst_p = jnp.concatenate([dst_s, jnp.full((EPK - EP,), N - 1, jnp.int32)])

    blk_lo = jnp.array([w * NBLK // NW for w in range(NW + 1)], jnp.int32)
    n_los = blk_lo * R
    bounds = jnp.searchsorted(dst_s, n_los).astype(jnp.int32)
    e_lo = bounds[:-1]
    e_hi = bounds[1:]
    e_start = e_lo - (e_lo % 8)
    nb = (e_hi - e_start + K - 1) // K
    meta = jnp.stack(
        [e_start, nb, n_los[:-1], n_los[1:], e_lo, e_hi,
         jnp.zeros((NW,), jnp.int32), jnp.zeros((NW,), jnp.int32)] +
        [jnp.zeros((NW,), jnp.int32)] * 8, axis=1).astype(jnp.int32)

    w1 = p['W1']
    as_ = [p['as1'], p['as2'], p['as3']]
    ad_ = [p['ad1'], p['ad2'], p['ad3']]
    ws = [w1, p['W2'], p['W3']]
    bs = [p['b1'].reshape(1, D), p['b2'].reshape(1, D), p['b3'].reshape(1, D)]

    xin = jnp.pad(x, ((0, N_BUF - N), (0, 0)))

    cur = xin
    for l in range(3):
        h_t, adst, gmax = _tc_layer(cur, ws[l], as_[l], ad_[l],
                                    bs[l - 1] if l else bs[0], l == 0)
        acc256, den16 = _sc_edge(h_t, adst, gmax, src_p, dst_p, meta)
        cur = jnp.concatenate([acc256, den16], axis=-1)

    bidx = jnp.pad(batch_idx, (0, N_BUF - N)).reshape(GRID, 1, BR)
    pwr = p['pw'].reshape(1, D)
    aw1 = jnp.concatenate([a['w1'] for a in p['adapters']], axis=1)
    ab1 = jnp.concatenate([a['b1'] for a in p['adapters']]).reshape(1, 160)
    aw2 = jnp.zeros((160, 8), f32)
    for k_ in range(5):
        aw2 = aw2.at[32 * k_:32 * (k_ + 1), k_].set(p['adapters'][k_]['w2'][:, 0])
    bw8 = jnp.pad(p['bw'], ((0, 0), (0, 7)))
    cw28 = jnp.pad(p['cw2'], ((0, 0), (0, 2)))
    bias = jnp.zeros((1, 128), f32)
    bias = bias.at[0, 0].set(p['bb'][0])
    for k_ in range(5):
        bias = bias.at[0, 8 + k_].set(p['adapters'][k_]['b2'][0])
    bias = bias.at[0, 16:22].set(p['cb2'])

    binp, enzp, clsp = _tc_tail(
        cur, bidx, bs[2], pwr, rnafm, hand_feat,
        p['e1w'], p['e1b'].reshape(1, 256), p['lng'].reshape(1, 256),
        p['lnb'].reshape(1, 256), p['e2w'], p['e2b'].reshape(1, 128),
        bw8, aw1, ab1, aw2, p['cw1'], p['cb1'].reshape(1, 64), cw28, bias)

    binary_logit = binp[:, 0]
    per_enzyme = tuple(enzp[:, k_] for k_ in range(5))
    cls = clsp[:, :6]
    return (binary_logit, per_enzyme, cls)
